# baseline (speedup 1.0000x reference)
import functools

import jax
import jax.numpy as jnp
from jax import lax
from jax.experimental import pallas as pl
from jax.experimental.pallas import tpu as pltpu
from jax.experimental.pallas import tpu_sc as plsc

N = 10000
E = 320000
NC, NS = 2, 16
NW = NC * NS
CHUNK = 128
CH = 80
E_PAD = NW * CH * CHUNK
TRASH = N
N_ACC = 10240
RPT = N_ACC // NS
TPT = N // NS
D1 = 16
K = 8
NB = CH // K


def _newton_rsqrt(x):
  y = 1.0 / x
  for _ in range(22):
    y = y * (1.5 - 0.5 * x * y * y)
  return y


def _fused_layer1():
  mesh = plsc.VectorSubcoreMesh(core_axis_name="c", subcore_axis_name="s")

  @functools.partial(
      pl.kernel,
      out_type=[
          jax.ShapeDtypeStruct((NC, N_ACC, D1), jnp.float32),
          jax.ShapeDtypeStruct((N_ACC,), jnp.float32),
      ],
      mesh=mesh,
      compiler_params=pltpu.CompilerParams(use_tc_tiling_on_sc=False),
      scratch_types=[
          pltpu.VMEM((CH, CHUNK), jnp.int32),
          pltpu.VMEM((NC, CH, CHUNK), jnp.int32),
          pltpu.VMEM((2, K, CHUNK, D1), jnp.float32),
          pltpu.VMEM((CHUNK,), jnp.float32),
          pltpu.VMEM((RPT, D1), jnp.float32),
          pltpu.VMEM((RPT,), jnp.float32),
          pltpu.SMEM((RPT,), jnp.float32),
          pltpu.VMEM_SHARED((N_ACC, D1), jnp.float32),
          pltpu.VMEM_SHARED((N_ACC, D1), jnp.float32),
          pltpu.VMEM_SHARED((N_ACC,), jnp.float32),
      ] + [pltpu.SemaphoreType.DMA] * 5,
  )
  def kern(src_hbm, dst_hbm, h1_hbm, ones_hbm, zeros_hbm, zeros1_hbm,
           agg_hbm, dinv_hbm,
           src_v, dst_v, rows_v, ones_v, tab_v, dslice_v, dinv_sm,
           tab_sh, acc_sh, deg_sh, g0, g1, s0, s1, t0):
    g_sem = (g0, g1)
    s_sem = (s0, s1)
    c = lax.axis_index("c")
    s = lax.axis_index("s")
    row0 = s * RPT
    pltpu.async_copy(h1_hbm.at[pl.ds(row0, RPT)], tab_v, t0)
    pltpu.sync_copy(zeros_hbm, acc_sh.at[pl.ds(row0, RPT)])
    pltpu.sync_copy(zeros1_hbm, deg_sh.at[pl.ds(row0, RPT)])
    pltpu.sync_copy(ones_hbm, ones_v)
    pltpu.sync_copy(src_hbm.at[c].at[s], src_v)
    pltpu.sync_copy(dst_hbm.at[0].at[s], dst_v.at[0])
    pltpu.sync_copy(dst_hbm.at[1].at[s], dst_v.at[1])
    plsc.subcore_barrier()

    def deg_fire(o, sem):
      def body(b, carry):
        for cc in range(NC):
          pltpu.async_copy(ones_v, deg_sh.at[dst_v.at[cc].at[o * 8 + b]],
                           sem, add=True)
        return carry
      lax.fori_loop(0, 8, body, 0)

    def deg_drain(o, sem):
      def body(b, carry):
        for cc in range(NC):
          pltpu.make_async_copy(ones_v,
                                deg_sh.at[dst_v.at[cc].at[o * 8 + b]],
                                sem).wait()
        return carry
      lax.fori_loop(0, 8, body, 0)

    deg_fire(0, g0)

    def deg_outer(m, carry):
      o = 2 * m
      deg_fire(o + 1, g1)
      deg_drain(o, g0)

      @pl.when(o + 2 < CH // 8)
      def _():
        deg_fire(o + 2, g0)
      deg_drain(o + 1, g1)
      return carry

    lax.fori_loop(0, CH // 16, deg_outer, 0)
    plsc.subcore_barrier()

    pltpu.sync_copy(deg_sh.at[pl.ds(row0, RPT)], dslice_v)

    def dinv_body(g, carry):
      deg = dslice_v[pl.ds(g * 16, 16)]
      dslice_v[pl.ds(g * 16, 16)] = _newton_rsqrt(deg + 1.0)
      return carry

    lax.fori_loop(0, RPT // 16, dinv_body, 0)

    pltpu.sync_copy(dslice_v, deg_sh.at[pl.ds(row0, RPT)])
    pltpu.sync_copy(deg_sh.at[pl.ds(row0, RPT)], dinv_sm)
    pltpu.make_async_copy(h1_hbm.at[pl.ds(row0, RPT)], tab_v, t0).wait()

    def scale_body(r, carry):
      tab_v[r, :] = tab_v[r, :] * dinv_sm[r]
      return carry

    lax.fori_loop(0, RPT, scale_body, 0)
    pltpu.sync_copy(tab_v, tab_sh.at[pl.ds(row0, RPT)])

    @pl.when(c == 0)
    def _():
      pltpu.sync_copy(dslice_v, dinv_hbm.at[pl.ds(row0, RPT)])
    plsc.subcore_barrier()

    def gather(j, buf, k, sem):
      pltpu.async_copy(tab_sh.at[src_v.at[j]], rows_v.at[buf].at[k], sem)

    def gather_wait(j, buf, k, sem):
      pltpu.make_async_copy(tab_sh.at[src_v.at[j]],
                            rows_v.at[buf].at[k], sem).wait()

    def scat(j, buf, k, sem):
      pltpu.async_copy(rows_v.at[buf].at[k], acc_sh.at[dst_v.at[c].at[j]],
                       sem, add=True)

    def scat_wait(j, buf, k, sem):
      pltpu.make_async_copy(rows_v.at[buf].at[k],
                            acc_sh.at[dst_v.at[c].at[j]], sem).wait()

    def loopk(fn, base, buf, sem):
      def body(k, carry):
        fn(base + k, buf, k, sem)
        return carry
      lax.fori_loop(0, K, body, 0)

    loopk(gather, 0, 0, g_sem[0])

    def block(o, buf):
      base = o * K
      loopk(gather_wait, base, buf, g_sem[buf])
      loopk(scat, base, buf, s_sem[buf])

      @pl.when(o > 0)
      def _():
        loopk(scat_wait, base - K, 1 - buf, s_sem[1 - buf])

      @pl.when(o + 1 < NB)
      def _():
        loopk(gather, base + K, 1 - buf, g_sem[1 - buf])

    def outer(m, carry):
      block(2 * m, 0)
      block(2 * m + 1, 1)
      return carry

    lax.fori_loop(0, NB // 2, outer, 0)
    loopk(scat_wait, CH - K, (NB - 1) % 2, s_sem[(NB - 1) % 2])
    plsc.subcore_barrier()
    pltpu.sync_copy(acc_sh.at[pl.ds(row0, RPT)],
                    agg_hbm.at[c].at[pl.ds(row0, RPT)])

  return kern


def _edge_pass(d_feat):
  mesh = plsc.VectorSubcoreMesh(core_axis_name="c", subcore_axis_name="s")

  @functools.partial(
      pl.kernel,
      out_type=jax.ShapeDtypeStruct((NC, N_ACC, d_feat), jnp.float32),
      mesh=mesh,
      compiler_params=pltpu.CompilerParams(use_tc_tiling_on_sc=False),
      scratch_types=[
          pltpu.VMEM((CH, CHUNK), jnp.int32),
          pltpu.VMEM((CH, CHUNK), jnp.int32),
          pltpu.VMEM((2, K, CHUNK, d_feat), jnp.float32),
          pltpu.VMEM_SHARED((N, d_feat), jnp.float32),
          pltpu.VMEM_SHARED((N_ACC, d_feat), jnp.float32),
      ] + [pltpu.SemaphoreType.DMA] * 4,
  )
  def kern(src_hbm, dst_hbm, table_hbm, zeros_hbm, out_hbm,
           src_v, dst_v, rows_v, tab_sh, acc_sh, g0, g1, s0, s1):
    g_sem = (g0, g1)
    s_sem = (s0, s1)
    c = lax.axis_index("c")
    s = lax.axis_index("s")
    row0 = s * RPT
    trow = s * TPT
    pltpu.sync_copy(table_hbm.at[pl.ds(trow, TPT)], tab_sh.at[pl.ds(trow, TPT)])
    pltpu.sync_copy(zeros_hbm, acc_sh.at[pl.ds(row0, RPT)])
    pltpu.sync_copy(src_hbm.at[c].at[s], src_v)
    pltpu.sync_copy(dst_hbm.at[c].at[s], dst_v)
    plsc.subcore_barrier()

    def gather(j, buf, k, sem):
      pltpu.async_copy(tab_sh.at[src_v.at[j]], rows_v.at[buf].at[k], sem)

    def gather_wait(j, buf, k, sem):
      pltpu.make_async_copy(tab_sh.at[src_v.at[j]],
                            rows_v.at[buf].at[k], sem).wait()

    def scat(j, buf, k, sem):
      pltpu.async_copy(rows_v.at[buf].at[k], acc_sh.at[dst_v.at[j]], sem,
                       add=True)

    def scat_wait(j, buf, k, sem):
      pltpu.make_async_copy(rows_v.at[buf].at[k],
                            acc_sh.at[dst_v.at[j]], sem).wait()

    def loopk(fn, base, buf, sem):
      def body(k, carry):
        fn(base + k, buf, k, sem)
        return carry
      lax.fori_loop(0, K, body, 0)

    loopk(gather, 0, 0, g_sem[0])

    def block(o, buf):
      base = o * K
      loopk(gather_wait, base, buf, g_sem[buf])
      loopk(scat, base, buf, s_sem[buf])

      @pl.when(o > 0)
      def _():
        loopk(scat_wait, base - K, 1 - buf, s_sem[1 - buf])

      @pl.when(o + 1 < NB)
      def _():
        loopk(gather, base + K, 1 - buf, g_sem[1 - buf])

    def outer(m, carry):
      block(2 * m, 0)
      block(2 * m + 1, 1)
      return carry

    lax.fori_loop(0, NB // 2, outer, 0)
    loopk(scat_wait, CH - K, (NB - 1) % 2, s_sem[(NB - 1) % 2])
    plsc.subcore_barrier()
    pltpu.sync_copy(acc_sh.at[pl.ds(row0, RPT)],
                    out_hbm.at[c].at[pl.ds(row0, RPT)])

  return kern


def _tc_a(x_ref, w1_ref, h1_ref):
  h1_ref[0:N, :] = jnp.dot(x_ref[...], w1_ref[...],
                           preferred_element_type=jnp.float32)
  h1_ref[N:N_ACC, :] = jnp.zeros((N_ACC - N, D1), jnp.float32)


def _tc_c(u2_ref, b2_ref, w2_ref, out_ref):
  z = (jnp.dot(u2_ref[...], w2_ref[...], preferred_element_type=jnp.float32)
       + b2_ref[...])
  m = jnp.max(z, axis=1, keepdims=True)
  lse = jnp.log(jnp.sum(jnp.exp(z - m), axis=1, keepdims=True))
  out_ref[...] = z - m - lse


def kernel(x, edge_index, W1, b1, W2, b2):
  ei = edge_index.astype(jnp.int32)
  pad = E_PAD - E
  pad_ids = jnp.arange(pad, dtype=jnp.int32)
  src = jnp.concatenate([ei[0], pad_ids % N])
  dst = jnp.concatenate([ei[1], TRASH + pad_ids % (N_ACC - N)])
  src = src.reshape(NC, NS, CH, CHUNK)
  dst = dst.reshape(NC, NS, CH, CHUNK)

  d2 = W2.shape[1]
  ones_c = jnp.ones((CHUNK,), jnp.float32)
  zeros_d1 = jnp.zeros((RPT, D1), jnp.float32)
  zeros_1 = jnp.zeros((RPT,), jnp.float32)

  h1 = pl.pallas_call(
      _tc_a,
      out_shape=jax.ShapeDtypeStruct((N_ACC, D1), jnp.float32),
  )(x, W1)

  agg1, dinv = _fused_layer1()(src, dst, h1, ones_c, zeros_d1, zeros_1)

  dinv2 = dinv[0:N][:, None]
  z1 = dinv2 * (agg1[0, 0:N, :] + agg1[1, 0:N, :] + h1[0:N, :] * dinv2)
  us = dinv2 * jnp.maximum(z1 + b1[None, :], 0.0)

  agg2 = _edge_pass(D1)(src, dst, us, zeros_d1)

  u2 = dinv2 * (agg2[0, 0:N, :] + agg2[1, 0:N, :] + us)
  out = pl.pallas_call(
      _tc_c,
      out_shape=jax.ShapeDtypeStruct((N, d2), jnp.float32),
  )(u2, b2[None, :], W2)

  return out


# speedup vs baseline: 1.0413x; 1.0413x over previous
import functools

import jax
import jax.numpy as jnp
from jax import lax
from jax.experimental import pallas as pl
from jax.experimental.pallas import tpu as pltpu
from jax.experimental.pallas import tpu_sc as plsc

N = 10000
E = 320000
NC, NS = 2, 16
NW = NC * NS
CHUNK = 128
CH = 80
E_PAD = NW * CH * CHUNK
TRASH = N
N_ACC = 10240
RPT = N_ACC // NS
TPT = N // NS
D1 = 16
K = 10
NB = CH // K


def _newton_rsqrt(x):
  y = 1.0 / x
  for _ in range(22):
    y = y * (1.5 - 0.5 * x * y * y)
  return y


def _fused_layer1():
  mesh = plsc.VectorSubcoreMesh(core_axis_name="c", subcore_axis_name="s")

  @functools.partial(
      pl.kernel,
      out_type=[
          jax.ShapeDtypeStruct((NC, N_ACC, D1), jnp.float32),
          jax.ShapeDtypeStruct((N_ACC,), jnp.float32),
      ],
      mesh=mesh,
      compiler_params=pltpu.CompilerParams(use_tc_tiling_on_sc=False),
      scratch_types=[
          pltpu.VMEM((CH, CHUNK), jnp.int32),
          pltpu.VMEM((NC, CH, CHUNK), jnp.int32),
          pltpu.VMEM((2, K, CHUNK, D1), jnp.float32),
          pltpu.VMEM((CHUNK,), jnp.float32),
          pltpu.VMEM((RPT, D1), jnp.float32),
          pltpu.VMEM((RPT,), jnp.float32),
          pltpu.SMEM((RPT,), jnp.float32),
          pltpu.VMEM_SHARED((N_ACC, D1), jnp.float32),
          pltpu.VMEM_SHARED((N_ACC, D1), jnp.float32),
          pltpu.VMEM_SHARED((N_ACC,), jnp.float32),
      ] + [pltpu.SemaphoreType.DMA] * 5,
  )
  def kern(src_hbm, dst_hbm, h1_hbm, ones_hbm, zeros_hbm, zeros1_hbm,
           agg_hbm, dinv_hbm,
           src_v, dst_v, rows_v, ones_v, tab_v, dslice_v, dinv_sm,
           tab_sh, acc_sh, deg_sh, g0, g1, s0, s1, t0):
    g_sem = (g0, g1)
    s_sem = (s0, s1)
    c = lax.axis_index("c")
    s = lax.axis_index("s")
    row0 = s * RPT
    pltpu.async_copy(h1_hbm.at[pl.ds(row0, RPT)], tab_v, t0)
    pltpu.sync_copy(zeros_hbm, acc_sh.at[pl.ds(row0, RPT)])
    pltpu.sync_copy(zeros1_hbm, deg_sh.at[pl.ds(row0, RPT)])
    pltpu.sync_copy(ones_hbm, ones_v)
    pltpu.sync_copy(src_hbm.at[c].at[s], src_v)
    pltpu.sync_copy(dst_hbm.at[0].at[s], dst_v.at[0])
    pltpu.sync_copy(dst_hbm.at[1].at[s], dst_v.at[1])
    plsc.subcore_barrier()

    def deg_fire(o, sem):
      def body(b, carry):
        for cc in range(NC):
          pltpu.async_copy(ones_v, deg_sh.at[dst_v.at[cc].at[o * 8 + b]],
                           sem, add=True)
        return carry
      lax.fori_loop(0, 8, body, 0)

    def deg_drain(o, sem):
      def body(b, carry):
        for cc in range(NC):
          pltpu.make_async_copy(ones_v,
                                deg_sh.at[dst_v.at[cc].at[o * 8 + b]],
                                sem).wait()
        return carry
      lax.fori_loop(0, 8, body, 0)

    deg_fire(0, g0)

    def deg_outer(m, carry):
      o = 2 * m
      deg_fire(o + 1, g1)
      deg_drain(o, g0)

      @pl.when(o + 2 < CH // 8)
      def _():
        deg_fire(o + 2, g0)
      deg_drain(o + 1, g1)
      return carry

    lax.fori_loop(0, CH // 16, deg_outer, 0)
    plsc.subcore_barrier()

    pltpu.sync_copy(deg_sh.at[pl.ds(row0, RPT)], dslice_v)

    def dinv_body(g, carry):
      deg = dslice_v[pl.ds(g * 16, 16)]
      dslice_v[pl.ds(g * 16, 16)] = _newton_rsqrt(deg + 1.0)
      return carry

    lax.fori_loop(0, RPT // 16, dinv_body, 0)

    pltpu.sync_copy(dslice_v, deg_sh.at[pl.ds(row0, RPT)])
    pltpu.sync_copy(deg_sh.at[pl.ds(row0, RPT)], dinv_sm)
    pltpu.make_async_copy(h1_hbm.at[pl.ds(row0, RPT)], tab_v, t0).wait()

    def scale_body(r, carry):
      tab_v[r, :] = tab_v[r, :] * dinv_sm[r]
      return carry

    lax.fori_loop(0, RPT, scale_body, 0)
    pltpu.sync_copy(tab_v, tab_sh.at[pl.ds(row0, RPT)])

    @pl.when(c == 0)
    def _():
      pltpu.sync_copy(dslice_v, dinv_hbm.at[pl.ds(row0, RPT)])
    plsc.subcore_barrier()

    def gather(j, buf, k, sem):
      pltpu.async_copy(tab_sh.at[src_v.at[j]], rows_v.at[buf].at[k], sem)

    def gather_wait(j, buf, k, sem):
      pltpu.make_async_copy(tab_sh.at[src_v.at[j]],
                            rows_v.at[buf].at[k], sem).wait()

    def scat(j, buf, k, sem):
      pltpu.async_copy(rows_v.at[buf].at[k], acc_sh.at[dst_v.at[c].at[j]],
                       sem, add=True)

    def scat_wait(j, buf, k, sem):
      pltpu.make_async_copy(rows_v.at[buf].at[k],
                            acc_sh.at[dst_v.at[c].at[j]], sem).wait()

    def loopk(fn, base, buf, sem):
      def body(k, carry):
        fn(base + k, buf, k, sem)
        return carry
      lax.fori_loop(0, K, body, 0)

    loopk(gather, 0, 0, g_sem[0])

    def block(o, buf):
      base = o * K
      loopk(gather_wait, base, buf, g_sem[buf])
      loopk(scat, base, buf, s_sem[buf])

      @pl.when(o > 0)
      def _():
        loopk(scat_wait, base - K, 1 - buf, s_sem[1 - buf])

      @pl.when(o + 1 < NB)
      def _():
        loopk(gather, base + K, 1 - buf, g_sem[1 - buf])

    def outer(m, carry):
      block(2 * m, 0)
      block(2 * m + 1, 1)
      return carry

    lax.fori_loop(0, NB // 2, outer, 0)
    loopk(scat_wait, CH - K, (NB - 1) % 2, s_sem[(NB - 1) % 2])
    plsc.subcore_barrier()
    pltpu.sync_copy(acc_sh.at[pl.ds(row0, RPT)],
                    agg_hbm.at[c].at[pl.ds(row0, RPT)])

  return kern


def _edge_pass(d_feat):
  mesh = plsc.VectorSubcoreMesh(core_axis_name="c", subcore_axis_name="s")

  @functools.partial(
      pl.kernel,
      out_type=jax.ShapeDtypeStruct((NC, N_ACC, d_feat), jnp.float32),
      mesh=mesh,
      compiler_params=pltpu.CompilerParams(use_tc_tiling_on_sc=False),
      scratch_types=[
          pltpu.VMEM((CH, CHUNK), jnp.int32),
          pltpu.VMEM((CH, CHUNK), jnp.int32),
          pltpu.VMEM((2, K, CHUNK, d_feat), jnp.float32),
          pltpu.VMEM_SHARED((N, d_feat), jnp.float32),
          pltpu.VMEM_SHARED((N_ACC, d_feat), jnp.float32),
      ] + [pltpu.SemaphoreType.DMA] * 4,
  )
  def kern(src_hbm, dst_hbm, table_hbm, zeros_hbm, out_hbm,
           src_v, dst_v, rows_v, tab_sh, acc_sh, g0, g1, s0, s1):
    g_sem = (g0, g1)
    s_sem = (s0, s1)
    c = lax.axis_index("c")
    s = lax.axis_index("s")
    row0 = s * RPT
    trow = s * TPT
    pltpu.sync_copy(table_hbm.at[pl.ds(trow, TPT)], tab_sh.at[pl.ds(trow, TPT)])
    pltpu.sync_copy(zeros_hbm, acc_sh.at[pl.ds(row0, RPT)])
    pltpu.sync_copy(src_hbm.at[c].at[s], src_v)
    pltpu.sync_copy(dst_hbm.at[c].at[s], dst_v)
    plsc.subcore_barrier()

    def gather(j, buf, k, sem):
      pltpu.async_copy(tab_sh.at[src_v.at[j]], rows_v.at[buf].at[k], sem)

    def gather_wait(j, buf, k, sem):
      pltpu.make_async_copy(tab_sh.at[src_v.at[j]],
                            rows_v.at[buf].at[k], sem).wait()

    def scat(j, buf, k, sem):
      pltpu.async_copy(rows_v.at[buf].at[k], acc_sh.at[dst_v.at[j]], sem,
                       add=True)

    def scat_wait(j, buf, k, sem):
      pltpu.make_async_copy(rows_v.at[buf].at[k],
                            acc_sh.at[dst_v.at[j]], sem).wait()

    def loopk(fn, base, buf, sem):
      def body(k, carry):
        fn(base + k, buf, k, sem)
        return carry
      lax.fori_loop(0, K, body, 0)

    loopk(gather, 0, 0, g_sem[0])

    def block(o, buf):
      base = o * K
      loopk(gather_wait, base, buf, g_sem[buf])
      loopk(scat, base, buf, s_sem[buf])

      @pl.when(o > 0)
      def _():
        loopk(scat_wait, base - K, 1 - buf, s_sem[1 - buf])

      @pl.when(o + 1 < NB)
      def _():
        loopk(gather, base + K, 1 - buf, g_sem[1 - buf])

    def outer(m, carry):
      block(2 * m, 0)
      block(2 * m + 1, 1)
      return carry

    lax.fori_loop(0, NB // 2, outer, 0)
    loopk(scat_wait, CH - K, (NB - 1) % 2, s_sem[(NB - 1) % 2])
    plsc.subcore_barrier()
    pltpu.sync_copy(acc_sh.at[pl.ds(row0, RPT)],
                    out_hbm.at[c].at[pl.ds(row0, RPT)])

  return kern


def _tc_a(x_ref, w1_ref, h1_ref):
  h1_ref[0:N, :] = jnp.dot(x_ref[...], w1_ref[...],
                           preferred_element_type=jnp.float32)
  h1_ref[N:N_ACC, :] = jnp.zeros((N_ACC - N, D1), jnp.float32)


def _tc_b(agg_ref, h1_ref, dinv_ref, b1_ref, us_ref):
  dinv = dinv_ref[0:N][:, None]
  h1s = h1_ref[0:N, :] * dinv
  z = dinv * (agg_ref[0, 0:N, :] + agg_ref[1, 0:N, :] + h1s) + b1_ref[...]
  us_ref[...] = dinv * jnp.maximum(z, 0.0)


def _tc_c(agg_ref, us_ref, dinv_ref, b2_ref, w2_ref, out_ref):
  u2 = (dinv_ref[0:N][:, None]
        * (agg_ref[0, 0:N, :] + agg_ref[1, 0:N, :] + us_ref[...]))
  z = jnp.dot(u2, w2_ref[...], preferred_element_type=jnp.float32) + b2_ref[...]
  m = jnp.max(z, axis=1, keepdims=True)
  lse = jnp.log(jnp.sum(jnp.exp(z - m), axis=1, keepdims=True))
  out_ref[...] = z - m - lse


def kernel(x, edge_index, W1, b1, W2, b2):
  ei = edge_index.astype(jnp.int32)
  pad = E_PAD - E
  pad_ids = jnp.arange(pad, dtype=jnp.int32)
  src = jnp.concatenate([ei[0], pad_ids % N])
  dst = jnp.concatenate([ei[1], TRASH + pad_ids % (N_ACC - N)])
  src = src.reshape(NC, NS, CH, CHUNK)
  dst = dst.reshape(NC, NS, CH, CHUNK)

  d2 = W2.shape[1]
  ones_c = jnp.ones((CHUNK,), jnp.float32)
  zeros_d1 = jnp.zeros((RPT, D1), jnp.float32)
  zeros_1 = jnp.zeros((RPT,), jnp.float32)

  h1 = pl.pallas_call(
      _tc_a,
      out_shape=jax.ShapeDtypeStruct((N_ACC, D1), jnp.float32),
  )(x, W1)

  agg1, dinv = _fused_layer1()(src, dst, h1, ones_c, zeros_d1, zeros_1)

  us = pl.pallas_call(
      _tc_b,
      out_shape=jax.ShapeDtypeStruct((N, D1), jnp.float32),
  )(agg1, h1, dinv, b1[None, :])

  agg2 = _edge_pass(D1)(src, dst, us, zeros_d1)

  out = pl.pallas_call(
      _tc_c,
      out_shape=jax.ShapeDtypeStruct((N, d2), jnp.float32),
  )(agg2, us, dinv, b2[None, :], W2)

  return out


# speedup vs baseline: 1.0776x; 1.0349x over previous
import functools

import jax
import jax.numpy as jnp
from jax import lax
from jax.experimental import pallas as pl
from jax.experimental.pallas import tpu as pltpu
from jax.experimental.pallas import tpu_sc as plsc

N = 10000
E = 320000
NC, NS = 2, 16
NW = NC * NS
CHUNK = 128
CH = 80
E_PAD = NW * CH * CHUNK
TRASH = N
N_ACC = 10240
RPT = N_ACC // NS
TPT = N // NS
D1 = 16
K = 10
NB = CH // K


def _newton_rsqrt(x):
  y = 1.0 / x
  for _ in range(22):
    y = y * (1.5 - 0.5 * x * y * y)
  return y


def _fused_layer1():
  mesh = plsc.VectorSubcoreMesh(core_axis_name="c", subcore_axis_name="s")

  @functools.partial(
      pl.kernel,
      out_type=[
          jax.ShapeDtypeStruct((NC, N_ACC, D1), jnp.float32),
          jax.ShapeDtypeStruct((N_ACC,), jnp.float32),
      ],
      mesh=mesh,
      compiler_params=pltpu.CompilerParams(use_tc_tiling_on_sc=False),
      scratch_types=[
          pltpu.VMEM((CH, CHUNK), jnp.int32),
          pltpu.VMEM((NC, CH, CHUNK), jnp.int32),
          pltpu.VMEM((2, K, CHUNK, D1), jnp.float32),
          pltpu.VMEM((CHUNK,), jnp.float32),
          pltpu.VMEM((RPT, D1), jnp.float32),
          pltpu.VMEM((RPT,), jnp.float32),
          pltpu.SMEM((RPT,), jnp.float32),
          pltpu.VMEM_SHARED((N_ACC, D1), jnp.float32),
          pltpu.VMEM_SHARED((N_ACC, D1), jnp.float32),
          pltpu.VMEM_SHARED((N_ACC,), jnp.float32),
      ] + [pltpu.SemaphoreType.DMA] * 5,
  )
  def kern(src_hbm, dst_hbm, h1_hbm, ones_hbm, zeros_hbm, zeros1_hbm,
           agg_hbm, dinv_hbm,
           src_v, dst_v, rows_v, ones_v, tab_v, dslice_v, dinv_sm,
           tab_sh, acc_sh, deg_sh, g0, g1, s0, s1, t0):
    g_sem = (g0, g1)
    s_sem = (s0, s1)
    c = lax.axis_index("c")
    s = lax.axis_index("s")
    row0 = s * RPT
    pltpu.async_copy(h1_hbm.at[pl.ds(row0, RPT)], tab_v, t0)
    pltpu.async_copy(zeros_hbm, acc_sh.at[pl.ds(row0, RPT)], s0)
    pltpu.async_copy(zeros1_hbm, deg_sh.at[pl.ds(row0, RPT)], s1)
    pltpu.async_copy(ones_hbm, ones_v, g0)
    pltpu.async_copy(src_hbm.at[c].at[s], src_v, g0)
    pltpu.async_copy(dst_hbm.at[0].at[s], dst_v.at[0], g1)
    pltpu.async_copy(dst_hbm.at[1].at[s], dst_v.at[1], g1)
    pltpu.make_async_copy(zeros_hbm, acc_sh.at[pl.ds(row0, RPT)], s0).wait()
    pltpu.make_async_copy(zeros1_hbm, deg_sh.at[pl.ds(row0, RPT)], s1).wait()
    pltpu.make_async_copy(ones_hbm, ones_v, g0).wait()
    pltpu.make_async_copy(src_hbm.at[c].at[s], src_v, g0).wait()
    pltpu.make_async_copy(dst_hbm.at[0].at[s], dst_v.at[0], g1).wait()
    pltpu.make_async_copy(dst_hbm.at[1].at[s], dst_v.at[1], g1).wait()
    plsc.subcore_barrier()

    def deg_fire(o, sem):
      def body(b, carry):
        for cc in range(NC):
          pltpu.async_copy(ones_v, deg_sh.at[dst_v.at[cc].at[o * 8 + b]],
                           sem, add=True)
        return carry
      lax.fori_loop(0, 8, body, 0)

    def deg_drain(o, sem):
      def body(b, carry):
        for cc in range(NC):
          pltpu.make_async_copy(ones_v,
                                deg_sh.at[dst_v.at[cc].at[o * 8 + b]],
                                sem).wait()
        return carry
      lax.fori_loop(0, 8, body, 0)

    deg_fire(0, g0)

    def deg_outer(m, carry):
      o = 2 * m
      deg_fire(o + 1, g1)
      deg_drain(o, g0)

      @pl.when(o + 2 < CH // 8)
      def _():
        deg_fire(o + 2, g0)
      deg_drain(o + 1, g1)
      return carry

    lax.fori_loop(0, CH // 16, deg_outer, 0)
    plsc.subcore_barrier()

    pltpu.sync_copy(deg_sh.at[pl.ds(row0, RPT)], dslice_v)

    def dinv_body(g, carry):
      deg = dslice_v[pl.ds(g * 16, 16)]
      dslice_v[pl.ds(g * 16, 16)] = _newton_rsqrt(deg + 1.0)
      return carry

    lax.fori_loop(0, RPT // 16, dinv_body, 0)

    pltpu.sync_copy(dslice_v, deg_sh.at[pl.ds(row0, RPT)])
    pltpu.sync_copy(deg_sh.at[pl.ds(row0, RPT)], dinv_sm)
    pltpu.make_async_copy(h1_hbm.at[pl.ds(row0, RPT)], tab_v, t0).wait()

    def scale_body(r, carry):
      tab_v[r, :] = tab_v[r, :] * dinv_sm[r]
      return carry

    lax.fori_loop(0, RPT, scale_body, 0)
    pltpu.sync_copy(tab_v, tab_sh.at[pl.ds(row0, RPT)])

    @pl.when(c == 0)
    def _():
      pltpu.sync_copy(dslice_v, dinv_hbm.at[pl.ds(row0, RPT)])
    plsc.subcore_barrier()

    def gather(j, buf, k, sem):
      pltpu.async_copy(tab_sh.at[src_v.at[j]], rows_v.at[buf].at[k], sem)

    def gather_wait(j, buf, k, sem):
      pltpu.make_async_copy(tab_sh.at[src_v.at[j]],
                            rows_v.at[buf].at[k], sem).wait()

    def scat(j, buf, k, sem):
      pltpu.async_copy(rows_v.at[buf].at[k], acc_sh.at[dst_v.at[c].at[j]],
                       sem, add=True)

    def scat_wait(j, buf, k, sem):
      pltpu.make_async_copy(rows_v.at[buf].at[k],
                            acc_sh.at[dst_v.at[c].at[j]], sem).wait()

    def loopk(fn, base, buf, sem):
      def body(k, carry):
        fn(base + k, buf, k, sem)
        return carry
      lax.fori_loop(0, K, body, 0)

    loopk(gather, 0, 0, g_sem[0])

    def block(o, buf):
      base = o * K
      loopk(gather_wait, base, buf, g_sem[buf])
      loopk(scat, base, buf, s_sem[buf])

      @pl.when(o > 0)
      def _():
        loopk(scat_wait, base - K, 1 - buf, s_sem[1 - buf])

      @pl.when(o + 1 < NB)
      def _():
        loopk(gather, base + K, 1 - buf, g_sem[1 - buf])

    def outer(m, carry):
      block(2 * m, 0)
      block(2 * m + 1, 1)
      return carry

    lax.fori_loop(0, NB // 2, outer, 0)
    loopk(scat_wait, CH - K, (NB - 1) % 2, s_sem[(NB - 1) % 2])
    plsc.subcore_barrier()
    pltpu.sync_copy(acc_sh.at[pl.ds(row0, RPT)],
                    agg_hbm.at[c].at[pl.ds(row0, RPT)])

  return kern


def _edge_pass(d_feat):
  mesh = plsc.VectorSubcoreMesh(core_axis_name="c", subcore_axis_name="s")

  @functools.partial(
      pl.kernel,
      out_type=jax.ShapeDtypeStruct((NC, N_ACC, d_feat), jnp.float32),
      mesh=mesh,
      compiler_params=pltpu.CompilerParams(use_tc_tiling_on_sc=False),
      scratch_types=[
          pltpu.VMEM((CH, CHUNK), jnp.int32),
          pltpu.VMEM((CH, CHUNK), jnp.int32),
          pltpu.VMEM((2, K, CHUNK, d_feat), jnp.float32),
          pltpu.VMEM_SHARED((N, d_feat), jnp.float32),
          pltpu.VMEM_SHARED((N_ACC, d_feat), jnp.float32),
      ] + [pltpu.SemaphoreType.DMA] * 4,
  )
  def kern(src_hbm, dst_hbm, table_hbm, zeros_hbm, out_hbm,
           src_v, dst_v, rows_v, tab_sh, acc_sh, g0, g1, s0, s1):
    g_sem = (g0, g1)
    s_sem = (s0, s1)
    c = lax.axis_index("c")
    s = lax.axis_index("s")
    row0 = s * RPT
    trow = s * TPT
    pltpu.async_copy(table_hbm.at[pl.ds(trow, TPT)],
                     tab_sh.at[pl.ds(trow, TPT)], g0)
    pltpu.async_copy(zeros_hbm, acc_sh.at[pl.ds(row0, RPT)], g1)
    pltpu.async_copy(src_hbm.at[c].at[s], src_v, s0)
    pltpu.async_copy(dst_hbm.at[c].at[s], dst_v, s1)
    pltpu.make_async_copy(table_hbm.at[pl.ds(trow, TPT)],
                          tab_sh.at[pl.ds(trow, TPT)], g0).wait()
    pltpu.make_async_copy(zeros_hbm, acc_sh.at[pl.ds(row0, RPT)], g1).wait()
    pltpu.make_async_copy(src_hbm.at[c].at[s], src_v, s0).wait()
    pltpu.make_async_copy(dst_hbm.at[c].at[s], dst_v, s1).wait()
    plsc.subcore_barrier()

    def gather(j, buf, k, sem):
      pltpu.async_copy(tab_sh.at[src_v.at[j]], rows_v.at[buf].at[k], sem)

    def gather_wait(j, buf, k, sem):
      pltpu.make_async_copy(tab_sh.at[src_v.at[j]],
                            rows_v.at[buf].at[k], sem).wait()

    def scat(j, buf, k, sem):
      pltpu.async_copy(rows_v.at[buf].at[k], acc_sh.at[dst_v.at[j]], sem,
                       add=True)

    def scat_wait(j, buf, k, sem):
      pltpu.make_async_copy(rows_v.at[buf].at[k],
                            acc_sh.at[dst_v.at[j]], sem).wait()

    def loopk(fn, base, buf, sem):
      def body(k, carry):
        fn(base + k, buf, k, sem)
        return carry
      lax.fori_loop(0, K, body, 0)

    loopk(gather, 0, 0, g_sem[0])

    def block(o, buf):
      base = o * K
      loopk(gather_wait, base, buf, g_sem[buf])
      loopk(scat, base, buf, s_sem[buf])

      @pl.when(o > 0)
      def _():
        loopk(scat_wait, base - K, 1 - buf, s_sem[1 - buf])

      @pl.when(o + 1 < NB)
      def _():
        loopk(gather, base + K, 1 - buf, g_sem[1 - buf])

    def outer(m, carry):
      block(2 * m, 0)
      block(2 * m + 1, 1)
      return carry

    lax.fori_loop(0, NB // 2, outer, 0)
    loopk(scat_wait, CH - K, (NB - 1) % 2, s_sem[(NB - 1) % 2])
    plsc.subcore_barrier()
    pltpu.sync_copy(acc_sh.at[pl.ds(row0, RPT)],
                    out_hbm.at[c].at[pl.ds(row0, RPT)])

  return kern


def _tc_a(x_ref, w1_ref, h1_ref):
  h1_ref[0:N, :] = jnp.dot(x_ref[...], w1_ref[...],
                           preferred_element_type=jnp.float32)
  h1_ref[N:N_ACC, :] = jnp.zeros((N_ACC - N, D1), jnp.float32)


def _tc_b(agg_ref, h1_ref, dinv_ref, b1_ref, us_ref):
  dinv = dinv_ref[0:N][:, None]
  h1s = h1_ref[0:N, :] * dinv
  z = dinv * (agg_ref[0, 0:N, :] + agg_ref[1, 0:N, :] + h1s) + b1_ref[...]
  us_ref[...] = dinv * jnp.maximum(z, 0.0)


def _tc_c(agg_ref, us_ref, dinv_ref, b2_ref, w2_ref, out_ref):
  u2 = (dinv_ref[0:N][:, None]
        * (agg_ref[0, 0:N, :] + agg_ref[1, 0:N, :] + us_ref[...]))
  z = jnp.dot(u2, w2_ref[...], preferred_element_type=jnp.float32) + b2_ref[...]
  m = jnp.max(z, axis=1, keepdims=True)
  lse = jnp.log(jnp.sum(jnp.exp(z - m), axis=1, keepdims=True))
  out_ref[...] = z - m - lse


def kernel(x, edge_index, W1, b1, W2, b2):
  ei = edge_index.astype(jnp.int32)
  pad = E_PAD - E
  pad_ids = jnp.arange(pad, dtype=jnp.int32)
  src = jnp.concatenate([ei[0], pad_ids % N])
  dst = jnp.concatenate([ei[1], TRASH + pad_ids % (N_ACC - N)])
  src = src.reshape(NC, NS, CH, CHUNK)
  dst = dst.reshape(NC, NS, CH, CHUNK)

  d2 = W2.shape[1]
  ones_c = jnp.ones((CHUNK,), jnp.float32)
  zeros_d1 = jnp.zeros((RPT, D1), jnp.float32)
  zeros_1 = jnp.zeros((RPT,), jnp.float32)

  h1 = pl.pallas_call(
      _tc_a,
      out_shape=jax.ShapeDtypeStruct((N_ACC, D1), jnp.float32),
  )(x, W1)

  agg1, dinv = _fused_layer1()(src, dst, h1, ones_c, zeros_d1, zeros_1)

  us = pl.pallas_call(
      _tc_b,
      out_shape=jax.ShapeDtypeStruct((N, D1), jnp.float32),
  )(agg1, h1, dinv, b1[None, :])

  agg2 = _edge_pass(D1)(src, dst, us, zeros_d1)

  out = pl.pallas_call(
      _tc_c,
      out_shape=jax.ShapeDtypeStruct((N, d2), jnp.float32),
  )(agg2, us, dinv, b2[None, :], W2)

  return out


# speedup vs baseline: 1.0783x; 1.0006x over previous
import functools

import jax
import jax.numpy as jnp
from jax import lax
from jax.experimental import pallas as pl
from jax.experimental.pallas import tpu as pltpu
from jax.experimental.pallas import tpu_sc as plsc

N = 10000
E = 320000
NC, NS = 2, 16
NW = NC * NS
CHUNK = 128
CH = 80
E_PAD = NW * CH * CHUNK
TRASH = N
N_ACC = 10240
RPT = N_ACC // NS
TPT = N // NS
D1 = 16
K = 10
NB = CH // K


def _newton_rsqrt(x):
  y = 1.0 / x
  for _ in range(22):
    y = y * (1.5 - 0.5 * x * y * y)
  return y


def _fused_layer1():
  mesh = plsc.VectorSubcoreMesh(core_axis_name="c", subcore_axis_name="s")

  @functools.partial(
      pl.kernel,
      out_type=[
          jax.ShapeDtypeStruct((NC, N_ACC, D1), jnp.float32),
          jax.ShapeDtypeStruct((N_ACC,), jnp.float32),
      ],
      mesh=mesh,
      compiler_params=pltpu.CompilerParams(use_tc_tiling_on_sc=False),
      scratch_types=[
          pltpu.VMEM((CH, CHUNK), jnp.int32),
          pltpu.VMEM((NC, CH, CHUNK), jnp.int32),
          pltpu.VMEM((2, K, CHUNK, D1), jnp.float32),
          pltpu.VMEM((CHUNK,), jnp.float32),
          pltpu.VMEM((RPT, D1), jnp.float32),
          pltpu.VMEM((RPT,), jnp.float32),
          pltpu.SMEM((RPT,), jnp.float32),
          pltpu.VMEM_SHARED((N_ACC, D1), jnp.float32),
          pltpu.VMEM_SHARED((N_ACC, D1), jnp.float32),
          pltpu.VMEM_SHARED((N_ACC,), jnp.float32),
      ] + [pltpu.SemaphoreType.DMA] * 5,
  )
  def kern(src_hbm, dst_hbm, h1_hbm, ones_hbm, zeros_hbm, zeros1_hbm,
           agg_hbm, dinv_hbm,
           src_v, dst_v, rows_v, ones_v, tab_v, dslice_v, dinv_sm,
           tab_sh, acc_sh, deg_sh, g0, g1, s0, s1, t0):
    g_sem = (g0, g1)
    s_sem = (s0, s1)
    c = lax.axis_index("c")
    s = lax.axis_index("s")
    row0 = s * RPT
    pltpu.async_copy(h1_hbm.at[pl.ds(row0, RPT)], tab_v, t0)
    pltpu.async_copy(zeros_hbm, acc_sh.at[pl.ds(row0, RPT)], s0)
    pltpu.async_copy(zeros1_hbm, deg_sh.at[pl.ds(row0, RPT)], s1)
    pltpu.async_copy(ones_hbm, ones_v, g0)
    pltpu.async_copy(src_hbm.at[c].at[s], src_v, g0)
    pltpu.async_copy(dst_hbm.at[0].at[s], dst_v.at[0], g1)
    pltpu.async_copy(dst_hbm.at[1].at[s], dst_v.at[1], g1)
    pltpu.make_async_copy(zeros_hbm, acc_sh.at[pl.ds(row0, RPT)], s0).wait()
    pltpu.make_async_copy(zeros1_hbm, deg_sh.at[pl.ds(row0, RPT)], s1).wait()
    pltpu.make_async_copy(ones_hbm, ones_v, g0).wait()
    pltpu.make_async_copy(src_hbm.at[c].at[s], src_v, g0).wait()
    pltpu.make_async_copy(dst_hbm.at[0].at[s], dst_v.at[0], g1).wait()
    pltpu.make_async_copy(dst_hbm.at[1].at[s], dst_v.at[1], g1).wait()
    plsc.subcore_barrier()

    def deg_fire(o, sem):
      def body(b, carry):
        for cc in range(NC):
          pltpu.async_copy(ones_v, deg_sh.at[dst_v.at[cc].at[o * 8 + b]],
                           sem, add=True)
        return carry
      lax.fori_loop(0, 8, body, 0)

    def deg_drain(o, sem):
      def body(b, carry):
        for cc in range(NC):
          pltpu.make_async_copy(ones_v,
                                deg_sh.at[dst_v.at[cc].at[o * 8 + b]],
                                sem).wait()
        return carry
      lax.fori_loop(0, 8, body, 0)

    deg_fire(0, g0)

    def deg_outer(m, carry):
      o = 2 * m
      deg_fire(o + 1, g1)
      deg_drain(o, g0)

      @pl.when(o + 2 < CH // 8)
      def _():
        deg_fire(o + 2, g0)
      deg_drain(o + 1, g1)
      return carry

    lax.fori_loop(0, CH // 16, deg_outer, 0)
    plsc.subcore_barrier()

    pltpu.sync_copy(deg_sh.at[pl.ds(row0, RPT)], dslice_v)

    def dinv_body(g, carry):
      deg = dslice_v[pl.ds(g * 16, 16)]
      dslice_v[pl.ds(g * 16, 16)] = _newton_rsqrt(deg + 1.0)
      return carry

    lax.fori_loop(0, RPT // 16, dinv_body, 0)

    pltpu.sync_copy(dslice_v, deg_sh.at[pl.ds(row0, RPT)])
    pltpu.sync_copy(deg_sh.at[pl.ds(row0, RPT)], dinv_sm)
    pltpu.make_async_copy(h1_hbm.at[pl.ds(row0, RPT)], tab_v, t0).wait()

    def scale_body(r, carry):
      tab_v[r, :] = tab_v[r, :] * dinv_sm[r]
      return carry

    lax.fori_loop(0, RPT, scale_body, 0)
    pltpu.sync_copy(tab_v, tab_sh.at[pl.ds(row0, RPT)])

    @pl.when(c == 0)
    def _():
      pltpu.async_copy(dslice_v, dinv_hbm.at[pl.ds(row0, RPT)], t0)
    plsc.subcore_barrier()

    def gather(j, buf, k, sem):
      pltpu.async_copy(tab_sh.at[src_v.at[j]], rows_v.at[buf].at[k], sem)

    def gather_wait(j, buf, k, sem):
      pltpu.make_async_copy(tab_sh.at[src_v.at[j]],
                            rows_v.at[buf].at[k], sem).wait()

    def scat(j, buf, k, sem):
      pltpu.async_copy(rows_v.at[buf].at[k], acc_sh.at[dst_v.at[c].at[j]],
                       sem, add=True)

    def scat_wait(j, buf, k, sem):
      pltpu.make_async_copy(rows_v.at[buf].at[k],
                            acc_sh.at[dst_v.at[c].at[j]], sem).wait()

    def loopk(fn, base, buf, sem):
      def body(k, carry):
        fn(base + k, buf, k, sem)
        return carry
      lax.fori_loop(0, K, body, 0)

    loopk(gather, 0, 0, g_sem[0])

    def block(o, buf):
      base = o * K
      loopk(gather_wait, base, buf, g_sem[buf])
      loopk(scat, base, buf, s_sem[buf])

      @pl.when(o > 0)
      def _():
        loopk(scat_wait, base - K, 1 - buf, s_sem[1 - buf])

      @pl.when(o + 1 < NB)
      def _():
        loopk(gather, base + K, 1 - buf, g_sem[1 - buf])

    def outer(m, carry):
      block(2 * m, 0)
      block(2 * m + 1, 1)
      return carry

    lax.fori_loop(0, NB // 2, outer, 0)
    loopk(scat_wait, CH - K, (NB - 1) % 2, s_sem[(NB - 1) % 2])

    @pl.when(c == 0)
    def _():
      pltpu.make_async_copy(dslice_v, dinv_hbm.at[pl.ds(row0, RPT)],
                            t0).wait()
    plsc.subcore_barrier()
    pltpu.sync_copy(acc_sh.at[pl.ds(row0, RPT)],
                    agg_hbm.at[c].at[pl.ds(row0, RPT)])

  return kern


def _edge_pass(d_feat):
  mesh = plsc.VectorSubcoreMesh(core_axis_name="c", subcore_axis_name="s")

  @functools.partial(
      pl.kernel,
      out_type=jax.ShapeDtypeStruct((NC, N_ACC, d_feat), jnp.float32),
      mesh=mesh,
      compiler_params=pltpu.CompilerParams(use_tc_tiling_on_sc=False),
      scratch_types=[
          pltpu.VMEM((CH, CHUNK), jnp.int32),
          pltpu.VMEM((CH, CHUNK), jnp.int32),
          pltpu.VMEM((2, K, CHUNK, d_feat), jnp.float32),
          pltpu.VMEM_SHARED((N, d_feat), jnp.float32),
          pltpu.VMEM_SHARED((N_ACC, d_feat), jnp.float32),
      ] + [pltpu.SemaphoreType.DMA] * 4,
  )
  def kern(src_hbm, dst_hbm, table_hbm, zeros_hbm, out_hbm,
           src_v, dst_v, rows_v, tab_sh, acc_sh, g0, g1, s0, s1):
    g_sem = (g0, g1)
    s_sem = (s0, s1)
    c = lax.axis_index("c")
    s = lax.axis_index("s")
    row0 = s * RPT
    trow = s * TPT
    pltpu.async_copy(table_hbm.at[pl.ds(trow, TPT)],
                     tab_sh.at[pl.ds(trow, TPT)], g0)
    pltpu.async_copy(zeros_hbm, acc_sh.at[pl.ds(row0, RPT)], g1)
    pltpu.async_copy(src_hbm.at[c].at[s], src_v, s0)
    pltpu.async_copy(dst_hbm.at[c].at[s], dst_v, s1)
    pltpu.make_async_copy(table_hbm.at[pl.ds(trow, TPT)],
                          tab_sh.at[pl.ds(trow, TPT)], g0).wait()
    pltpu.make_async_copy(zeros_hbm, acc_sh.at[pl.ds(row0, RPT)], g1).wait()
    pltpu.make_async_copy(src_hbm.at[c].at[s], src_v, s0).wait()
    pltpu.make_async_copy(dst_hbm.at[c].at[s], dst_v, s1).wait()
    plsc.subcore_barrier()

    def gather(j, buf, k, sem):
      pltpu.async_copy(tab_sh.at[src_v.at[j]], rows_v.at[buf].at[k], sem)

    def gather_wait(j, buf, k, sem):
      pltpu.make_async_copy(tab_sh.at[src_v.at[j]],
                            rows_v.at[buf].at[k], sem).wait()

    def scat(j, buf, k, sem):
      pltpu.async_copy(rows_v.at[buf].at[k], acc_sh.at[dst_v.at[j]], sem,
                       add=True)

    def scat_wait(j, buf, k, sem):
      pltpu.make_async_copy(rows_v.at[buf].at[k],
                            acc_sh.at[dst_v.at[j]], sem).wait()

    def loopk(fn, base, buf, sem):
      def body(k, carry):
        fn(base + k, buf, k, sem)
        return carry
      lax.fori_loop(0, K, body, 0)

    loopk(gather, 0, 0, g_sem[0])

    def block(o, buf):
      base = o * K
      loopk(gather_wait, base, buf, g_sem[buf])
      loopk(scat, base, buf, s_sem[buf])

      @pl.when(o > 0)
      def _():
        loopk(scat_wait, base - K, 1 - buf, s_sem[1 - buf])

      @pl.when(o + 1 < NB)
      def _():
        loopk(gather, base + K, 1 - buf, g_sem[1 - buf])

    def outer(m, carry):
      block(2 * m, 0)
      block(2 * m + 1, 1)
      return carry

    lax.fori_loop(0, NB // 2, outer, 0)
    loopk(scat_wait, CH - K, (NB - 1) % 2, s_sem[(NB - 1) % 2])
    plsc.subcore_barrier()
    pltpu.sync_copy(acc_sh.at[pl.ds(row0, RPT)],
                    out_hbm.at[c].at[pl.ds(row0, RPT)])

  return kern


def _tc_a(x_ref, w1_ref, h1_ref):
  h1_ref[0:N, :] = jnp.dot(x_ref[...], w1_ref[...],
                           preferred_element_type=jnp.float32)
  h1_ref[N:N_ACC, :] = jnp.zeros((N_ACC - N, D1), jnp.float32)


def _tc_b(agg_ref, h1_ref, dinv_ref, b1_ref, us_ref):
  dinv = dinv_ref[0:N][:, None]
  h1s = h1_ref[0:N, :] * dinv
  z = dinv * (agg_ref[0, 0:N, :] + agg_ref[1, 0:N, :] + h1s) + b1_ref[...]
  us_ref[...] = dinv * jnp.maximum(z, 0.0)


def _tc_c(agg_ref, us_ref, dinv_ref, b2_ref, w2_ref, out_ref):
  u2 = (dinv_ref[0:N][:, None]
        * (agg_ref[0, 0:N, :] + agg_ref[1, 0:N, :] + us_ref[...]))
  z = jnp.dot(u2, w2_ref[...], preferred_element_type=jnp.float32) + b2_ref[...]
  m = jnp.max(z, axis=1, keepdims=True)
  lse = jnp.log(jnp.sum(jnp.exp(z - m), axis=1, keepdims=True))
  out_ref[...] = z - m - lse


def kernel(x, edge_index, W1, b1, W2, b2):
  ei = edge_index.astype(jnp.int32)
  pad = E_PAD - E
  pad_ids = jnp.arange(pad, dtype=jnp.int32)
  src = jnp.concatenate([ei[0], pad_ids % N])
  dst = jnp.concatenate([ei[1], TRASH + pad_ids % (N_ACC - N)])
  src = src.reshape(NC, NS, CH, CHUNK)
  dst = dst.reshape(NC, NS, CH, CHUNK)

  d2 = W2.shape[1]
  ones_c = jnp.ones((CHUNK,), jnp.float32)
  zeros_d1 = jnp.zeros((RPT, D1), jnp.float32)
  zeros_1 = jnp.zeros((RPT,), jnp.float32)

  h1 = pl.pallas_call(
      _tc_a,
      out_shape=jax.ShapeDtypeStruct((N_ACC, D1), jnp.float32),
  )(x, W1)

  agg1, dinv = _fused_layer1()(src, dst, h1, ones_c, zeros_d1, zeros_1)

  us = pl.pallas_call(
      _tc_b,
      out_shape=jax.ShapeDtypeStruct((N, D1), jnp.float32),
  )(agg1, h1, dinv, b1[None, :])

  agg2 = _edge_pass(D1)(src, dst, us, zeros_d1)

  out = pl.pallas_call(
      _tc_c,
      out_shape=jax.ShapeDtypeStruct((N, d2), jnp.float32),
  )(agg2, us, dinv, b2[None, :], W2)

  return out
